# baseline (speedup 1.0000x reference)
import jax
import jax.numpy as jnp
from jax import lax
from jax.experimental import pallas as pl
from jax.experimental.pallas import tpu as pltpu
from jax.experimental.pallas import tpu_sc as plsc

MAX_LEN = 200
EMB_DIM = 32
BATCH = 4096
N_IDX = BATCH * MAX_LEN
N_GRP = MAX_LEN // 4
N_ROWS = N_IDX // 4
VOCAB = 1000000
TROWS = VOCAB // 4

_info = plsc.get_sparse_core_info()
NC, NS = _info.num_cores, _info.num_subcores
NW = NC * NS

TCHUNK = 768
N_TCHUNKS = 999936 // TCHUNK
REM = VOCAB - N_TCHUNKS * TCHUNK
TPAIRS = (N_TCHUNKS // NW + 2) // 2


def _transpose_compute(in_b, out_v, b):
    lane = lax.iota(jnp.int32, 16)

    @plsc.parallel_loop(0, TCHUNK // 4, unroll=4)
    def _(r):
        for q in range(4):
            j = 4 * r + q
            for h in range(2):
                v = plsc.load_gather(in_b, [16 * h + lane, lane * 0 + j])
                out_v[b, r, pl.ds(q * EMB_DIM + 16 * h, 16)] = v


def _transpose_body(tt_hbm, tail_hbm, out_hbm, in_v0, in_v1, out_v, sin, sout):
    wid = lax.axis_index("s") * NC + lax.axis_index("c")
    myn = (N_TCHUNKS - 1 - wid) // NW + 1
    in_bufs = (in_v0, in_v1)

    def in_desc(j, b):
        c = j * NW + wid
        return pltpu.make_async_copy(
            tt_hbm.at[:, pl.ds(c * TCHUNK, TCHUNK)],
            in_bufs[b].at[:, pl.ds(0, TCHUNK)],
            sin,
        )

    def out_desc(j, b):
        c = j * NW + wid
        return pltpu.make_async_copy(
            out_v.at[b],
            out_hbm.at[pl.ds(c * (TCHUNK // 4), TCHUNK // 4)],
            sout,
        )

    in_desc(0, 0).start()

    def pair_body(j2, carry):
        for b in (0, 1):
            j = j2 * 2 + b

            @pl.when(j < myn)
            def _():
                in_desc(j, b).wait()

                @pl.when(j + 1 < myn)
                def _():
                    in_desc(j + 1, 1 - b).start()

                @pl.when(j >= 2)
                def _():
                    out_desc(j - 2, b).wait()

                _transpose_compute(in_bufs[b], out_v, b)
                out_desc(j, b).start()

        return carry

    lax.fori_loop(0, TPAIRS, pair_body, 0)
    out_desc(0, 0).wait()
    out_desc(0, 1).wait()

    @pl.when(wid == 17)
    def _():
        pltpu.sync_copy(tail_hbm, out_v.at[0, pl.ds(0, REM // 4)])
        pltpu.sync_copy(
            out_v.at[0, pl.ds(0, REM // 4)],
            out_hbm.at[pl.ds((VOCAB - REM) // 4, REM // 4)],
        )


def _sc_transpose(tableT, tail128):
    mesh = plsc.VectorSubcoreMesh(core_axis_name="c", subcore_axis_name="s")
    kern = pl.kernel(
        _transpose_body,
        mesh=mesh,
        out_type=jax.ShapeDtypeStruct((TROWS, 128), jnp.float32),
        scratch_types=[
            pltpu.VMEM((EMB_DIM, TCHUNK + 1), jnp.float32),
            pltpu.VMEM((EMB_DIM, TCHUNK + 1), jnp.float32),
            pltpu.VMEM((2, TCHUNK // 4, 128), jnp.float32),
            pltpu.SemaphoreType.DMA,
            pltpu.SemaphoreType.DMA,
        ],
        compiler_params=pltpu.CompilerParams(
            use_tc_tiling_on_sc=True, needs_layout_passes=False
        ),
    )
    return kern(tableT, tail128)


BPW = BATCH // NW
TPC = 8
GITER = MAX_LEN // TPC
GN = TPC * BPW


def _gather_body(x_hbm, table_hbm, out_hbm, idx2_v, idxi_v, rows_v, sem):
    wid = lax.axis_index("s") * NC + lax.axis_index("c")
    lane = lax.iota(jnp.int32, 16)
    b0 = wid * BPW

    def chunk_body(i, carry):
        pltpu.sync_copy(
            x_hbm.at[pl.ds(i * TPC, TPC), pl.ds(b0, BPW)],
            idx2_v.at[:, pl.ds(0, BPW)],
        )

        @plsc.parallel_loop(0, GN // 16, unroll=2)
        def _(g2):
            n0 = g2 * 16
            kh = n0 >> 9
            rvec = kh * 4 + (lane & 3)
            cvec = ((n0 & 511) >> 2) + (lane >> 2)
            idxi_v[pl.ds(n0, 16)] = plsc.load_gather(idx2_v, [rvec, cvec])

        pltpu.async_copy(table_hbm.at[idxi_v], rows_v, sem).wait()
        for kh in range(2):
            k = 2 * i + kh
            pltpu.sync_copy(
                rows_v.at[pl.ds(kh * 512, 512)],
                out_hbm.at[pl.ds(k * (4 * BATCH) + 4 * b0, 512)],
            )
        return carry

    lax.fori_loop(0, GITER, chunk_body, 0)


def _sc_gather(x2d, table_lin):
    mesh = plsc.VectorSubcoreMesh(core_axis_name="c", subcore_axis_name="s")
    kern = pl.kernel(
        _gather_body,
        mesh=mesh,
        out_type=jax.ShapeDtypeStruct((N_IDX, EMB_DIM), jnp.float32),
        scratch_types=[
            pltpu.VMEM((TPC, BPW + 4), jnp.int32),
            pltpu.VMEM((GN,), jnp.int32),
            pltpu.VMEM((GN, EMB_DIM), jnp.float32),
            pltpu.SemaphoreType.DMA,
        ],
        compiler_params=pltpu.CompilerParams(
            use_tc_tiling_on_sc=False, needs_layout_passes=False
        ),
    )
    return kern(x2d, table_lin)


KPER = 5
NSTEP = N_GRP // KPER


def _mlp_body(emb_ref, w1_ref, b1_ref, w2_ref, b2_ref, out_ref, acc_ref):
    k0 = pl.program_id(0)

    @pl.when(k0 == 0)
    def _():
        acc_ref[...] = jnp.zeros_like(acc_ref)

    acc = acc_ref[...]
    for s in range(KPER):
        acc += jnp.dot(
            emb_ref[pl.ds(s * BATCH, BATCH), :],
            w1_ref[pl.ds(s * 128, 128), :],
            preferred_element_type=jnp.float32,
        )
    acc_ref[...] = acc

    @pl.when(k0 == NSTEP - 1)
    def _():
        h = jnp.maximum(acc + b1_ref[...], 0.0)
        o = jnp.dot(h, w2_ref[...], preferred_element_type=jnp.float32)
        out_ref[...] = jax.nn.sigmoid(o + b2_ref[...])


def _tc_mlp(emb128, w1t, b1, w2t, b2):
    f = pl.pallas_call(
        _mlp_body,
        grid=(NSTEP,),
        in_specs=[
            pl.BlockSpec((KPER * BATCH, 128), lambda k0: (k0, 0)),
            pl.BlockSpec((KPER * 128, 32), lambda k0: (k0, 0)),
            pl.BlockSpec((1, 32), lambda k0: (0, 0)),
            pl.BlockSpec((32, 1), lambda k0: (0, 0)),
            pl.BlockSpec((1, 1), lambda k0: (0, 0)),
        ],
        out_specs=pl.BlockSpec((BATCH, 1), lambda k0: (0, 0)),
        out_shape=jax.ShapeDtypeStruct((BATCH, 1), jnp.float32),
        scratch_shapes=[pltpu.VMEM((BATCH, 32), jnp.float32)],
    )
    return f(emb128, w1t, b1, w2t, b2)


@jax.jit
def kernel(x, table, W1, b1, W2, b2):
    tail128 = table[VOCAB - REM :].reshape(REM // 4, 128)
    tbl128 = _sc_transpose(table.T, tail128)
    table_lin = tbl128.reshape(VOCAB, EMB_DIM)
    emb128 = _sc_gather(x.T, table_lin).reshape(N_ROWS, 128)
    return _tc_mlp(emb128, W1.T, b1.reshape(1, 32), W2.T, b2.reshape(1, 1))
